# baseline (speedup 1.0000x reference)
import functools

import jax
import jax.numpy as jnp
from jax import lax
from jax.experimental import pallas as pl
from jax.experimental.pallas import tpu as pltpu
from jax.experimental.pallas import tpu_sc as plsc

_N = 16777216
_NC = 2
_NS = 16
_L = 16
_NW = _NC * _NS
_PER_W = _N // _NW
_CHUNK = 65536
_NDMA = _PER_W // _CHUNK


@functools.partial(
    pl.kernel,
    mesh=plsc.VectorSubcoreMesh(core_axis_name="c", subcore_axis_name="s"),
    out_type=jax.ShapeDtypeStruct((_N,), jnp.float32),
    scratch_types=[
        pltpu.VMEM((_CHUNK,), jnp.float32),
        pltpu.VMEM((_L,), jnp.int32),
        pltpu.VMEM((_L,), jnp.float32),
        pltpu.VMEM((_L,), jnp.float32),
        pltpu.SemaphoreType.DMA,
    ],
)
def _sc_fill(idx_hbm, grad_hbm, out_hbm, zbuf, ivec, gvec, gtile, sem):
    c = lax.axis_index("c")
    s = lax.axis_index("s")
    wid = s * _NC + c
    base = wid * _PER_W

    zeros16 = jnp.zeros((_L,), jnp.float32)

    def _zero_body(i, carry):
        zbuf[pl.ds(i * _L, _L)] = zeros16
        return carry

    lax.fori_loop(0, _CHUNK // _L, _zero_body, 0)

    copies = [
        pltpu.make_async_copy(
            zbuf, out_hbm.at[pl.ds(base + j * _CHUNK, _CHUNK)], sem)
        for j in range(_NDMA)
    ]
    for cp in copies:
        cp.start()
    for cp in copies:
        cp.wait()

    pltpu.sync_copy(idx_hbm, ivec)
    pltpu.sync_copy(grad_hbm, gvec)
    sidx = ivec[...][0]

    @pl.when(sidx // _PER_W == wid)
    def _():
        aligned = jnp.minimum((sidx // 8) * 8, base + _PER_W - _L)
        off = sidx - aligned
        lanes = lax.iota(jnp.int32, _L)
        gtile[...] = jnp.where(lanes == off, gvec[...], 0.0)
        pltpu.sync_copy(gtile, out_hbm.at[pl.ds(aligned, _L)])


def kernel(grad_output, input_sizes, dim, index, out):
    del out
    idx = ((jnp.asarray(index, jnp.int32) + jnp.asarray(dim, jnp.int32))
           % jnp.asarray(input_sizes, jnp.int32))
    idx_b = jnp.broadcast_to(idx, (_L,))
    grad_b = jnp.broadcast_to(jnp.asarray(grad_output, jnp.float32), (_L,))
    return _sc_fill(idx_b, grad_b)


# speedup vs baseline: 1.3622x; 1.3622x over previous
import functools

import jax
import jax.numpy as jnp
from jax import lax
from jax.experimental import pallas as pl
from jax.experimental.pallas import tpu as pltpu
from jax.experimental.pallas import tpu_sc as plsc

_N = 16777216
_NC = 2
_NS = 16
_L = 16
_NW = _NC * _NS
_PER_W = _N // _NW
_CHUNK = 65536
_NDMA = _PER_W // _CHUNK


@functools.partial(
    pl.kernel,
    mesh=plsc.VectorSubcoreMesh(core_axis_name="c", subcore_axis_name="s"),
    out_type=jax.ShapeDtypeStruct((_N,), jnp.float32),
    scratch_types=[
        pltpu.VMEM((_CHUNK,), jnp.float32),
        pltpu.VMEM((_L,), jnp.int32),
        pltpu.VMEM((_L,), jnp.float32),
        pltpu.VMEM((_L,), jnp.float32),
        pltpu.SemaphoreType.DMA,
    ],
)
def _sc_fill(idx_hbm, grad_hbm, out_hbm, zbuf, ivec, gvec, gtile, sem):
    c = lax.axis_index("c")
    s = lax.axis_index("s")
    wid = s * _NC + c
    base = wid * _PER_W

    zeros16 = jnp.zeros((_L,), jnp.float32)

    _UNROLL = 16

    def _zero_body(i, carry):
        for j in range(_UNROLL):
            zbuf[pl.ds((i * _UNROLL + j) * _L, _L)] = zeros16
        return carry

    lax.fori_loop(0, _CHUNK // (_L * _UNROLL), _zero_body, 0)

    copies = [
        pltpu.make_async_copy(
            zbuf, out_hbm.at[pl.ds(base + j * _CHUNK, _CHUNK)], sem)
        for j in range(_NDMA)
    ]
    for cp in copies:
        cp.start()
    for cp in copies:
        cp.wait()

    pltpu.sync_copy(idx_hbm, ivec)
    pltpu.sync_copy(grad_hbm, gvec)
    sidx = ivec[...][0]

    @pl.when(sidx // _PER_W == wid)
    def _():
        aligned = jnp.minimum((sidx // 8) * 8, base + _PER_W - _L)
        off = sidx - aligned
        lanes = lax.iota(jnp.int32, _L)
        gtile[...] = jnp.where(lanes == off, gvec[...], 0.0)
        pltpu.sync_copy(gtile, out_hbm.at[pl.ds(aligned, _L)])


def kernel(grad_output, input_sizes, dim, index, out):
    del out
    idx = ((jnp.asarray(index, jnp.int32) + jnp.asarray(dim, jnp.int32))
           % jnp.asarray(input_sizes, jnp.int32))
    idx_b = jnp.broadcast_to(idx, (_L,))
    grad_b = jnp.broadcast_to(jnp.asarray(grad_output, jnp.float32), (_L,))
    return _sc_fill(idx_b, grad_b)
